# baseline (speedup 1.0000x reference)
import functools

import jax
import jax.numpy as jnp
from jax import lax
from jax.experimental import pallas as pl
from jax.experimental.pallas import tpu as pltpu
from jax.experimental.pallas import tpu_sc as plsc

N_OUT = 10000
ACC_PAD = 10240
NC, NS = 2, 16
RB = 2560
CH = 80


def _mm_body(e_ref, w_ref, b_ref, h_ref):
    h = jax.lax.dot_general(
        e_ref[...].astype(jnp.bfloat16), w_ref[...],
        (((1,), (1,)), ((), ())),
        preferred_element_type=jnp.float32)
    h_ref[...] = jnp.maximum(h + b_ref[...], 0.0)


def _sc_body(h_hbm, idx_hbm, z_hbm, out_hbm, idx_v, rows_v, acc_sh):
    c = lax.axis_index("c")
    s = lax.axis_index("s")
    n_e = h_hbm.shape[0]
    rows_per_tile = n_e // (NC * NS)
    nbatch = rows_per_tile // CH
    seg_slice = ACC_PAD // NS

    pltpu.sync_copy(z_hbm.at[pl.ds(s * seg_slice, seg_slice)],
                    acc_sh.at[pl.ds(s * seg_slice, seg_slice)])
    plsc.subcore_barrier()

    base = (c * NS + s) * rows_per_tile

    def body(j, carry):
        rb = base + j * CH
        pltpu.sync_copy(idx_hbm.at[pl.ds(rb, CH)], idx_v)
        pltpu.sync_copy(h_hbm.at[pl.ds(rb, CH)], rows_v)
        pltpu.sync_copy(rows_v, acc_sh.at[idx_v], add=True)
        return carry

    lax.fori_loop(0, nbatch, body, 0)
    plsc.subcore_barrier()

    pltpu.sync_copy(acc_sh.at[pl.ds(s * seg_slice, seg_slice)],
                    out_hbm.at[c, pl.ds(s * seg_slice, seg_slice)])


def _merge_body(p_ref, o_ref):
    o_ref[...] = p_ref[0] + p_ref[1]


def kernel(e, index, W, b):
    n_e, d = e.shape
    nb = n_e // RB
    b2 = b.reshape(1, d)

    h = pl.pallas_call(
        _mm_body,
        grid=(nb,),
        in_specs=[
            pl.BlockSpec((RB, d), lambda i: (i, 0)),
            pl.BlockSpec((d, d), lambda i: (0, 0)),
            pl.BlockSpec((1, d), lambda i: (0, 0)),
        ],
        out_specs=pl.BlockSpec((RB, d), lambda i: (i, 0)),
        out_shape=jax.ShapeDtypeStruct((n_e, d), jnp.float32),
    )(e, W.astype(jnp.bfloat16), b2)

    zeros = jnp.zeros((ACC_PAD, d), jnp.float32)
    mesh = plsc.VectorSubcoreMesh(core_axis_name="c", subcore_axis_name="s")
    partials = pl.kernel(
        _sc_body,
        out_type=jax.ShapeDtypeStruct((NC, ACC_PAD, d), jnp.float32),
        mesh=mesh,
        scratch_types=[
            pltpu.VMEM((CH,), jnp.int32),
            pltpu.VMEM((CH, d), jnp.float32),
            pltpu.VMEM_SHARED((ACC_PAD, d), jnp.float32),
        ],
    )(h, index, zeros)

    mb = 2000
    return pl.pallas_call(
        _merge_body,
        grid=(N_OUT // mb,),
        in_specs=[pl.BlockSpec((NC, mb, d), lambda i: (0, i, 0))],
        out_specs=pl.BlockSpec((mb, d), lambda i: (i, 0)),
        out_shape=jax.ShapeDtypeStruct((N_OUT, d), jnp.float32),
    )(partials)


# speedup vs baseline: 1.7496x; 1.7496x over previous
import functools

import jax
import jax.numpy as jnp
from jax import lax
from jax.experimental import pallas as pl
from jax.experimental.pallas import tpu as pltpu
from jax.experimental.pallas import tpu_sc as plsc

N_OUT = 10000
ACC_PAD = 10240
NC, NS = 2, 16
RB = 8000
CH = 80


def _mm_body(e_ref, w_ref, b_ref, h_ref):
    h = jax.lax.dot_general(
        e_ref[...].astype(jnp.bfloat16), w_ref[...],
        (((1,), (1,)), ((), ())),
        preferred_element_type=jnp.float32)
    h_ref[...] = jnp.maximum(h + b_ref[...], 0.0)


def _sc_body(h_hbm, idx_hbm, z_hbm, out_hbm,
             idx0, idx1, rows0, rows1, acc_sh, sem0, sem1):
    c = lax.axis_index("c")
    s = lax.axis_index("s")
    n_e = h_hbm.shape[0]
    rows_per_tile = n_e // (NC * NS)
    nbatch = rows_per_tile // CH
    seg_slice = ACC_PAD // NS

    pltpu.sync_copy(z_hbm.at[pl.ds(s * seg_slice, seg_slice)],
                    acc_sh.at[pl.ds(s * seg_slice, seg_slice)])
    plsc.subcore_barrier()

    base = (c * NS + s) * rows_per_tile

    pltpu.async_copy(idx_hbm.at[pl.ds(base, CH)], idx0, sem0)
    pltpu.async_copy(h_hbm.at[pl.ds(base, CH)], rows0, sem0)

    def step(j, carry):
        even = lax.rem(j, 2) == 0
        off = base + j * CH
        off2 = off + CH

        @pl.when(j + 1 < nbatch)
        def _issue_next():
            @pl.when(even)
            def _():
                pltpu.async_copy(idx_hbm.at[pl.ds(off2, CH)], idx1, sem1)
                pltpu.async_copy(h_hbm.at[pl.ds(off2, CH)], rows1, sem1)

            @pl.when(jnp.logical_not(even))
            def _():
                pltpu.async_copy(idx_hbm.at[pl.ds(off2, CH)], idx0, sem0)
                pltpu.async_copy(h_hbm.at[pl.ds(off2, CH)], rows0, sem0)

        @pl.when(even)
        def _consume0():
            pltpu.make_async_copy(idx_hbm.at[pl.ds(off, CH)], idx0, sem0).wait()
            pltpu.make_async_copy(h_hbm.at[pl.ds(off, CH)], rows0, sem0).wait()
            pltpu.sync_copy(rows0, acc_sh.at[idx0], add=True)

        @pl.when(jnp.logical_not(even))
        def _consume1():
            pltpu.make_async_copy(idx_hbm.at[pl.ds(off, CH)], idx1, sem1).wait()
            pltpu.make_async_copy(h_hbm.at[pl.ds(off, CH)], rows1, sem1).wait()
            pltpu.sync_copy(rows1, acc_sh.at[idx1], add=True)

        return carry

    lax.fori_loop(0, nbatch, step, 0)
    plsc.subcore_barrier()

    pltpu.sync_copy(acc_sh.at[pl.ds(s * seg_slice, seg_slice)],
                    out_hbm.at[c, pl.ds(s * seg_slice, seg_slice)])


def _merge_body(p_ref, o_ref):
    o_ref[...] = p_ref[0] + p_ref[1]


def kernel(e, index, W, b):
    n_e, d = e.shape
    nb = n_e // RB
    b2 = b.reshape(1, d)

    h = pl.pallas_call(
        _mm_body,
        grid=(nb,),
        in_specs=[
            pl.BlockSpec((RB, d), lambda i: (i, 0)),
            pl.BlockSpec((d, d), lambda i: (0, 0)),
            pl.BlockSpec((1, d), lambda i: (0, 0)),
        ],
        out_specs=pl.BlockSpec((RB, d), lambda i: (i, 0)),
        out_shape=jax.ShapeDtypeStruct((n_e, d), jnp.float32),
    )(e, W.astype(jnp.bfloat16), b2)

    zeros = jnp.zeros((ACC_PAD, d), jnp.float32)
    mesh = plsc.VectorSubcoreMesh(core_axis_name="c", subcore_axis_name="s")
    partials = pl.kernel(
        _sc_body,
        out_type=jax.ShapeDtypeStruct((NC, ACC_PAD, d), jnp.float32),
        mesh=mesh,
        scratch_types=[
            pltpu.VMEM((CH,), jnp.int32),
            pltpu.VMEM((CH,), jnp.int32),
            pltpu.VMEM((CH, d), jnp.float32),
            pltpu.VMEM((CH, d), jnp.float32),
            pltpu.VMEM_SHARED((ACC_PAD, d), jnp.float32),
            pltpu.SemaphoreType.DMA,
            pltpu.SemaphoreType.DMA,
        ],
    )(h, index, zeros)

    mb = 2000
    return pl.pallas_call(
        _merge_body,
        grid=(N_OUT // mb,),
        in_specs=[pl.BlockSpec((NC, mb, d), lambda i: (0, i, 0))],
        out_specs=pl.BlockSpec((mb, d), lambda i: (i, 0)),
        out_shape=jax.ShapeDtypeStruct((N_OUT, d), jnp.float32),
    )(partials)
